# Initial kernel scaffold; baseline (speedup 1.0000x reference)
import jax
import jax.numpy as jnp
from jax.experimental import pallas as pl


def kernel(feat0, feat1, feat2, pos, W_fc0, b_fc0, W_fc1, b_fc1, W_fc2, b_fc2, attn_l1, attn_r1, attn_l2, attn_r2, W_att_sc, b_att_sc, a_sc, W_g1, b_g1, pa1, W_g2, b_g2, pa2, W_att_mp, b_att_mp, a_mp, W_p1, b_p1, W_p2, b_p2, bg1_src, bg1_dst, bg2_src, bg2_dst, mg1_src, mg1_dst, mg2_src, mg2_dst):
    raise NotImplementedError("write your pallas kernel here")


# speedup vs baseline: 6.2200x; 6.2200x over previous
import functools

import jax
import jax.numpy as jnp
from jax import lax
from jax.experimental import pallas as pl
from jax.experimental.pallas import tpu as pltpu
from jax.experimental.pallas import tpu_sc as plsc

_N = 10000
_H = 64
_K1 = 10
_K2 = 5
_TAU = 0.8
_LAM = 0.5


def _elu(x):
    return jnp.where(x > 0, x, jnp.exp(x) - 1.0)


def _fc_body(x_ref, w_ref, b_ref, e_ref, o_ref):
    blk = x_ref.shape[0]
    h = jnp.dot(x_ref[...], w_ref[...], preferred_element_type=jnp.float32)
    h = _elu(h + b_ref[...])
    o_ref[...] = jnp.concatenate(
        [h, jnp.zeros((blk, 128 - _H), jnp.float32)], axis=1) + e_ref[...]


def _fc_elu(x, w_t, b, extra):
    m, d = x.shape
    blk = 2000
    return pl.pallas_call(
        _fc_body,
        grid=(m // blk,),
        in_specs=[
            pl.BlockSpec((blk, d), lambda i: (i, 0)),
            pl.BlockSpec((d, _H), lambda i: (0, 0)),
            pl.BlockSpec((1, _H), lambda i: (0, 0)),
            pl.BlockSpec((1, 128), lambda i: (0, 0)),
        ],
        out_specs=pl.BlockSpec((blk, 128), lambda i: (i, 0)),
        out_shape=jax.ShapeDtypeStruct((m, 128), jnp.float32),
    )(x, w_t, b.reshape(1, _H), extra)


_CHUNK = 128


def _sc_gat_gather(h1, idx1, h2, idx2):
    e1p, e2p = idx1.shape[0], idx2.shape[0]
    n1 = e1p // (32 * _CHUNK)
    n2 = e2p // (32 * _CHUNK)
    mesh = plsc.VectorSubcoreMesh(core_axis_name="c", subcore_axis_name="s")

    @functools.partial(
        pl.kernel,
        out_type=(
            jax.ShapeDtypeStruct((e1p, 128), jnp.float32),
            jax.ShapeDtypeStruct((e2p, 128), jnp.float32),
        ),
        mesh=mesh,
        scratch_types=[
            pltpu.VMEM((_CHUNK,), jnp.int32),
            pltpu.VMEM((_CHUNK, 128), jnp.float32),
            pltpu.SemaphoreType.DMA,
        ],
    )
    def k(h1_hbm, i1_hbm, h2_hbm, i2_hbm, o1_hbm, o2_hbm, idx_v, rows_v, sem):
        wid = lax.axis_index("s") * 2 + lax.axis_index("c")

        def run(tab, ih, oh, nchunk):
            base = wid * (nchunk * _CHUNK)

            def body(ci, _):
                off = base + ci * _CHUNK
                pltpu.sync_copy(ih.at[pl.ds(off, _CHUNK)], idx_v)
                pltpu.async_copy(tab.at[idx_v], rows_v, sem).wait()
                pltpu.sync_copy(rows_v, oh.at[pl.ds(off, _CHUNK)])
                return 0

            lax.fori_loop(0, nchunk, body, 0)

        run(h1_hbm, i1_hbm, o1_hbm, n1)
        run(h2_hbm, i2_hbm, o2_hbm, n2)

    return k(h1, idx1, h2, idx2)


_ACC_ROWS = 10240
_TRASH = 10200
_WID_E = 10112


def _sc_gcn_acc(h0e, s1, d1, s2, d2):
    nchunk = _WID_E // _CHUNK
    mesh = plsc.VectorSubcoreMesh(core_axis_name="c", subcore_axis_name="s")

    @functools.partial(
        pl.kernel,
        out_type=(
            jax.ShapeDtypeStruct((2, _ACC_ROWS, 128), jnp.float32),
            jax.ShapeDtypeStruct((2, _ACC_ROWS, 128), jnp.float32),
        ),
        mesh=mesh,
        scratch_types=[
            pltpu.VMEM((_CHUNK,), jnp.int32),
            pltpu.VMEM((_CHUNK,), jnp.int32),
            pltpu.VMEM((_CHUNK, 128), jnp.float32),
            pltpu.VMEM_SHARED((_ACC_ROWS, 128), jnp.float32),
            pltpu.SemaphoreType.DMA,
        ],
    )
    def k(tab, s1h, d1h, s2h, d2h, o1, o2, idx_s, idx_d, rows_v, acc, sem):
        cid = lax.axis_index("c")
        sid = lax.axis_index("s")
        wid = sid * 2 + cid

        def zero_acc():
            def zrow(r, _):
                for kk in range(8):
                    rows_v[r, pl.ds(kk * 16, 16)] = jnp.zeros((16,), jnp.float32)
                return 0

            lax.fori_loop(0, _CHUNK, zrow, 0)
            for p in range(5):
                pltpu.sync_copy(rows_v, acc.at[pl.ds(sid * 640 + p * _CHUNK, _CHUNK)])

        for (sh, dh, oh) in ((s1h, d1h, o1), (s2h, d2h, o2)):
            zero_acc()
            plsc.subcore_barrier()
            base = wid * _WID_E

            def body(ci, _, sh=sh, dh=dh):
                off = base + ci * _CHUNK
                pltpu.sync_copy(sh.at[pl.ds(off, _CHUNK)], idx_s)
                pltpu.sync_copy(dh.at[pl.ds(off, _CHUNK)], idx_d)
                pltpu.async_copy(tab.at[idx_s], rows_v, sem).wait()
                pltpu.sync_copy(rows_v, acc.at[idx_d], add=True)
                return 0

            lax.fori_loop(0, nchunk, body, 0)
            plsc.subcore_barrier()
            pltpu.sync_copy(
                acc.at[pl.ds(sid * 640, 640)], oh.at[cid, pl.ds(sid * 640, 640)]
            )
            plsc.subcore_barrier()

    return k(h0e, s1, d1, s2, d2)


_BLK = 1000


def _d1_body(h0_ref, ge1_ref, ge2_ref, a1_ref, a2_ref,
             al1_ref, ar1_ref, al2_ref, ar2_ref,
             wg1_ref, bg1_ref, pa1_ref, wg2_ref, bg2_ref, pa2_ref,
             wsc_ref, bsc_ref, wmp_ref, bmp_ref,
             z1_ref, z2_ref, g1_ref, g2_ref, t_ref):
    h0b = h0_ref[:, :_H]

    def gat(ge, al, ar, kk):
        er = jnp.sum(h0b * ar, axis=1, keepdims=True)
        es = []
        for k in range(kk):
            gk = ge[:, 128 * k:128 * k + _H]
            e = jnp.sum(gk * al, axis=1, keepdims=True) + er
            es.append(jnp.where(e > 0, e, 0.01 * e))
        m = functools.reduce(jnp.maximum, es)
        num = jnp.zeros((_BLK, _H), jnp.float32)
        den = jnp.zeros((_BLK, 1), jnp.float32)
        for k in range(kk):
            ex = jnp.exp(es[k] - m)
            num = num + ex * ge[:, 128 * k:128 * k + _H]
            den = den + ex
        return _elu(num / (den + 1e-16))

    z1 = gat(ge1_ref[...], al1_ref[...], ar1_ref[...], _K1)
    z2 = gat(ge2_ref[...], al2_ref[...], ar2_ref[...], _K2)
    z1_ref[...] = z1
    z2_ref[...] = z2

    def gcn(a, wg, bg, pa):
        s = a[0] + a[1]
        h = s[:, :_H] / jnp.maximum(s[:, _H:_H + 1], 1.0)
        hg = jnp.dot(h, wg, preferred_element_type=jnp.float32) + bg
        return jnp.where(hg >= 0, hg, pa * hg)

    g1 = gcn(a1_ref[...], wg1_ref[...], bg1_ref[...], pa1_ref[0, 0])
    g2 = gcn(a2_ref[...], wg2_ref[...], bg2_ref[...], pa2_ref[0, 0])
    g1_ref[...] = g1
    g2_ref[...] = g2

    def tfeat(z, w, b):
        t = jnp.tanh(jnp.dot(z, w, preferred_element_type=jnp.float32) + b)
        return jnp.sum(t, axis=0, keepdims=True)

    wsc, bsc = wsc_ref[...], bsc_ref[...]
    wmp, bmp = wmp_ref[...], bmp_ref[...]
    tt = jnp.concatenate(
        [tfeat(z1, wsc, bsc), tfeat(z2, wsc, bsc),
         tfeat(g1, wmp, bmp), tfeat(g2, wmp, bmp)], axis=0)

    @pl.when(pl.program_id(0) == 0)
    def _():
        t_ref[...] = tt

    @pl.when(pl.program_id(0) != 0)
    def _():
        t_ref[...] = t_ref[...] + tt


def _d1(h0, ge1, ge2, a1, a2, al1, ar1, al2, ar2,
        wg1, bg1, pa1, wg2, bg2, pa2, wsc, bsc, wmp, bmp):
    vec = lambda v: v.reshape(1, _H)
    full = lambda shp: pl.BlockSpec(shp, lambda i: tuple(0 for _ in shp))
    grid = _N // _BLK
    return pl.pallas_call(
        _d1_body,
        grid=(grid,),
        in_specs=[
            pl.BlockSpec((_BLK, 128), lambda i: (i, 0)),
            pl.BlockSpec((_BLK, 128 * _K1), lambda i: (i, 0)),
            pl.BlockSpec((_BLK, 128 * _K2), lambda i: (i, 0)),
            pl.BlockSpec((2, _BLK, 128), lambda i: (0, i, 0)),
            pl.BlockSpec((2, _BLK, 128), lambda i: (0, i, 0)),
            full((1, _H)), full((1, _H)), full((1, _H)), full((1, _H)),
            full((_H, _H)), full((1, _H)), full((1, 1)),
            full((_H, _H)), full((1, _H)), full((1, 1)),
            full((_H, _H)), full((1, _H)),
            full((_H, _H)), full((1, _H)),
        ],
        out_specs=[
            pl.BlockSpec((_BLK, _H), lambda i: (i, 0)),
            pl.BlockSpec((_BLK, _H), lambda i: (i, 0)),
            pl.BlockSpec((_BLK, _H), lambda i: (i, 0)),
            pl.BlockSpec((_BLK, _H), lambda i: (i, 0)),
            pl.BlockSpec((4, _H), lambda i: (0, 0)),
        ],
        out_shape=[
            jax.ShapeDtypeStruct((_N, _H), jnp.float32),
            jax.ShapeDtypeStruct((_N, _H), jnp.float32),
            jax.ShapeDtypeStruct((_N, _H), jnp.float32),
            jax.ShapeDtypeStruct((_N, _H), jnp.float32),
            jax.ShapeDtypeStruct((4, _H), jnp.float32),
        ],
        compiler_params=pltpu.CompilerParams(
            dimension_semantics=("arbitrary",)),
    )(h0, ge1, ge2, a1, a2, vec(al1), vec(ar1), vec(al2), vec(ar2),
      wg1, vec(bg1), pa1.reshape(1, 1), wg2, vec(bg2), pa2.reshape(1, 1),
      wsc, vec(bsc), wmp, vec(bmp))


def _d2_body(t_ref, asc_ref, amp_ref, z1_ref, z2_ref, g1_ref, g2_ref,
             wp1_ref, bp1_ref, wp2_ref, bp2_ref, psc_ref, pmp_ref):
    t = t_ref[...]

    def betas(t_a, t_b, a_vec):
        w1 = jnp.sum(t_a * a_vec) / _N
        w2 = jnp.sum(t_b * a_vec) / _N
        m = jnp.maximum(w1, w2)
        e1 = jnp.exp(w1 - m)
        e2 = jnp.exp(w2 - m)
        return e1 / (e1 + e2), e2 / (e1 + e2)

    b1, b2 = betas(t[0:1], t[1:2], asc_ref[...])
    c1, c2 = betas(t[2:3], t[3:4], amp_ref[...])

    wp1, bp1 = wp1_ref[...], bp1_ref[...]
    wp2, bp2 = wp2_ref[...], bp2_ref[...]

    def proj(z):
        p = _elu(jnp.dot(z, wp1, preferred_element_type=jnp.float32) + bp1)
        return jnp.dot(p, wp2, preferred_element_type=jnp.float32) + bp2

    p = proj(b1 * z1_ref[...] + b2 * z2_ref[...])
    xn = jnp.sqrt(jnp.sum(p * p, axis=1, keepdims=True))
    psc_ref[...] = p / (xn * _TAU)

    q = proj(c1 * g1_ref[...] + c2 * g2_ref[...])
    yn = jnp.sqrt(jnp.sum(q * q, axis=1, keepdims=True))
    pmp_ref[...] = q / yn


def _d2(t, a_sc, a_mp, z1, z2, g1, g2, wp1t, bp1, wp2t, bp2):
    vec = lambda v: v.reshape(1, _H)
    full = lambda shp: pl.BlockSpec(shp, lambda i: tuple(0 for _ in shp))
    blkspec = pl.BlockSpec((_BLK, _H), lambda i: (i, 0))
    return pl.pallas_call(
        _d2_body,
        grid=(_N // _BLK,),
        in_specs=[full((4, _H)), full((1, _H)), full((1, _H)),
                  blkspec, blkspec, blkspec, blkspec,
                  full((_H, _H)), full((1, _H)), full((_H, _H)), full((1, _H))],
        out_specs=[blkspec, blkspec],
        out_shape=[
            jax.ShapeDtypeStruct((_N, _H), jnp.float32),
            jax.ShapeDtypeStruct((_N, _H), jnp.float32),
        ],
    )(t, vec(a_sc), vec(a_mp), z1, z2, g1, g2, wp1t, vec(bp1), wp2t, vec(bp2))


_BE = 200


def _e_body(psc_i_ref, pmp_ref, pmp_i_ref, psc_ref, pos_ref,
            rs_ref, rsp_ref, cs_ref, csp_ref):
    dn = (((1,), (1,)), ((), ()))
    posb = pos_ref[...]
    sa = jnp.exp(lax.dot_general(psc_i_ref[...], pmp_ref[...], dn,
                                 preferred_element_type=jnp.float32))
    rs_ref[...] = jnp.sum(sa, axis=1, keepdims=True)
    rsp_ref[...] = jnp.sum(sa * posb, axis=1, keepdims=True)
    sb = jnp.exp(lax.dot_general(pmp_i_ref[...], psc_ref[...], dn,
                                 preferred_element_type=jnp.float32))
    cs_ref[...] = jnp.sum(sb, axis=1, keepdims=True)
    csp_ref[...] = jnp.sum(sb * posb, axis=1, keepdims=True)


def _e_pass(psc, pmp, pos):
    rowspec = pl.BlockSpec((_BE, _H), lambda i: (i, 0))
    fullspec = pl.BlockSpec((_N, _H), lambda i: (0, 0))
    outspec = pl.BlockSpec((_BE, 1), lambda i: (i, 0))
    out1 = jax.ShapeDtypeStruct((_N, 1), jnp.float32)
    return pl.pallas_call(
        _e_body,
        grid=(_N // _BE,),
        in_specs=[rowspec, fullspec, rowspec, fullspec,
                  pl.BlockSpec((_BE, _N), lambda i: (i, 0))],
        out_specs=[outspec, outspec, outspec, outspec],
        out_shape=[out1, out1, out1, out1],
        compiler_params=pltpu.CompilerParams(
            dimension_semantics=("arbitrary",)),
    )(psc, pmp, pmp, psc, pos)


def _f_body(rs_ref, rsp_ref, cs_ref, csp_ref, o_ref):
    ls = jnp.log(rs_ref[...] + 1e-8) - jnp.log(rsp_ref[...])
    lm = jnp.log(cs_ref[...] + 1e-8) - jnp.log(csp_ref[...])
    o_ref[...] = (_LAM * jnp.mean(ls) + (1.0 - _LAM) * jnp.mean(lm)).reshape(1, 1)


def _f_pass(rs, rsp, cs, csp):
    spec = pl.BlockSpec((_N, 1), lambda: (0, 0))
    return pl.pallas_call(
        _f_body,
        in_specs=[spec, spec, spec, spec],
        out_specs=pl.BlockSpec((1, 1), lambda: (0, 0)),
        out_shape=jax.ShapeDtypeStruct((1, 1), jnp.float32),
    )(rs, rsp, cs, csp)


def _pad_idx(idx, total, fill):
    return jnp.pad(idx, (0, total - idx.shape[0]), constant_values=fill)


def kernel(feat0, feat1, feat2, pos, W_fc0, b_fc0, W_fc1, b_fc1, W_fc2, b_fc2,
           attn_l1, attn_r1, attn_l2, attn_r2, W_att_sc, b_att_sc, a_sc,
           W_g1, b_g1, pa1, W_g2, b_g2, pa2, W_att_mp, b_att_mp, a_mp,
           W_p1, b_p1, W_p2, b_p2, bg1_src, bg1_dst, bg2_src, bg2_dst,
           mg1_src, mg1_dst, mg2_src, mg2_dst):
    ones_col = jnp.zeros((1, 128), jnp.float32).at[0, _H].set(1.0)
    zero_col = jnp.zeros((1, 128), jnp.float32)
    h0e = _fc_elu(feat0, W_fc0.T, b_fc0, ones_col)
    h1 = _fc_elu(feat1, W_fc1.T, b_fc1, zero_col)
    h2 = _fc_elu(feat2, W_fc2.T, b_fc2, zero_col)

    e1p = 32 * _CHUNK * 25
    e2p = 32 * _CHUNK * 14
    ge1, ge2 = _sc_gat_gather(
        h1, _pad_idx(bg1_src, e1p, 0), h2, _pad_idx(bg2_src, e2p, 0))
    ge1 = ge1[:_N * _K1].reshape(_N, _K1 * 128)
    ge2 = ge2[:_N * _K2].reshape(_N, _K2 * 128)

    emp = 32 * _WID_E
    a1, a2 = _sc_gcn_acc(
        h0e,
        _pad_idx(mg1_src, emp, 0), _pad_idx(mg1_dst, emp, _TRASH),
        _pad_idx(mg2_src, emp, 0), _pad_idx(mg2_dst, emp, _TRASH))
    a1 = a1[:, :_N, :]
    a2 = a2[:, :_N, :]

    z1, z2, g1, g2, t = _d1(
        h0e, ge1, ge2, a1, a2, attn_l1, attn_r1, attn_l2, attn_r2,
        W_g1, b_g1, pa1, W_g2, b_g2, pa2, W_att_sc, b_att_sc,
        W_att_mp, b_att_mp)

    psc, pmp = _d2(t, a_sc, a_mp, z1, z2, g1, g2, W_p1.T, b_p1, W_p2.T, b_p2)

    rs, rsp, cs, csp = _e_pass(psc, pmp, pos)
    loss = _f_pass(rs, rsp, cs, csp)
    return loss.reshape(())
